# initial kernel scaffold (unmeasured)
import jax
import jax.numpy as jnp
from jax import lax
from jax.experimental import pallas as pl
from jax.experimental.pallas import tpu as pltpu


def kernel(
    x,
):
    def body(*refs):
        pass

    out_shape = jax.ShapeDtypeStruct(..., jnp.float32)
    return pl.pallas_call(body, out_shape=out_shape)(...)


# baseline (device time: 53473 ns/iter reference)
import jax
import jax.numpy as jnp
from jax import lax
from jax.experimental import pallas as pl
from jax.experimental.pallas import tpu as pltpu


def kernel(x):
    m, n = x.shape

    def body(x_ref, out_ref, recv_ref, send_sem, recv_sem):
        my_x = lax.axis_index("x")
        my_y = lax.axis_index("y")
        my_z = lax.axis_index("z")
        partner = (1 - my_x, my_y, my_z)

        barrier_sem = pltpu.get_barrier_semaphore()
        pl.semaphore_signal(
            barrier_sem, inc=1,
            device_id=partner, device_id_type=pl.DeviceIdType.MESH,
        )
        pl.semaphore_wait(barrier_sem, 1)

        rdma = pltpu.make_async_remote_copy(
            src_ref=x_ref,
            dst_ref=recv_ref,
            send_sem=send_sem,
            recv_sem=recv_sem,
            device_id=partner,
            device_id_type=pl.DeviceIdType.MESH,
        )
        rdma.start()
        rdma.wait()
        out_ref[...] = x_ref[...] + recv_ref[...]

    return pl.pallas_call(
        body,
        out_shape=jax.ShapeDtypeStruct((m, n), x.dtype),
        in_specs=[pl.BlockSpec(memory_space=pltpu.VMEM)],
        out_specs=pl.BlockSpec(memory_space=pltpu.VMEM),
        scratch_shapes=[
            pltpu.VMEM((m, n), x.dtype),
            pltpu.SemaphoreType.DMA,
            pltpu.SemaphoreType.DMA,
        ],
        compiler_params=pltpu.CompilerParams(collective_id=0),
    )(x)


# device time: 36087 ns/iter; 1.4818x vs baseline; 1.4818x over previous
import jax
import jax.numpy as jnp
from jax import lax
from jax.experimental import pallas as pl
from jax.experimental.pallas import tpu as pltpu

C = 8


def kernel(x):
    m, n = x.shape
    half = m // 2
    rows = half // C

    def body(x_ref, out_ref, recv_ref, x_send, x_recv, z_send, z_recv):
        my_x = lax.axis_index("x")
        my_y = lax.axis_index("y")
        my_z = lax.axis_index("z")
        xp = (1 - my_x, my_y, my_z)
        zp = (my_x, my_y, 1 - my_z)
        base = my_z * half

        barrier_sem = pltpu.get_barrier_semaphore()
        for nbr in (xp, zp):
            pl.semaphore_signal(
                barrier_sem, inc=1,
                device_id=nbr, device_id_type=pl.DeviceIdType.MESH,
            )
        pl.semaphore_wait(barrier_sem, 2)

        x_rdmas = []
        for c in range(C):
            r = pltpu.make_async_remote_copy(
                src_ref=x_ref.at[pl.ds(base + c * rows, rows), :],
                dst_ref=recv_ref.at[pl.ds(c * rows, rows), :],
                send_sem=x_send.at[c],
                recv_sem=x_recv.at[c],
                device_id=xp,
                device_id_type=pl.DeviceIdType.MESH,
            )
            r.start()
            x_rdmas.append(r)

        z_rdmas = []
        for c in range(C):
            x_rdmas[c].wait_recv()
            sl = pl.ds(base + c * rows, rows)
            out_ref[sl, :] = x_ref[sl, :] + recv_ref[pl.ds(c * rows, rows), :]
            rz = pltpu.make_async_remote_copy(
                src_ref=out_ref.at[sl, :],
                dst_ref=out_ref.at[sl, :],
                send_sem=z_send.at[c],
                recv_sem=z_recv.at[c],
                device_id=zp,
                device_id_type=pl.DeviceIdType.MESH,
            )
            rz.start()
            z_rdmas.append(rz)

        for c in range(C):
            z_rdmas[c].wait_recv()
            x_rdmas[c].wait_send()
            z_rdmas[c].wait_send()

    return pl.pallas_call(
        body,
        out_shape=jax.ShapeDtypeStruct((m, n), x.dtype),
        in_specs=[pl.BlockSpec(memory_space=pltpu.VMEM)],
        out_specs=pl.BlockSpec(memory_space=pltpu.VMEM),
        scratch_shapes=[
            pltpu.VMEM((half, n), x.dtype),
            pltpu.SemaphoreType.DMA((C,)),
            pltpu.SemaphoreType.DMA((C,)),
            pltpu.SemaphoreType.DMA((C,)),
            pltpu.SemaphoreType.DMA((C,)),
        ],
        compiler_params=pltpu.CompilerParams(collective_id=0),
    )(x)


# device time: 29379 ns/iter; 1.8201x vs baseline; 1.2283x over previous
import jax
import jax.numpy as jnp
from jax import lax
from jax.experimental import pallas as pl
from jax.experimental.pallas import tpu as pltpu

C = 8
H = C // 2


def kernel(x):
    m, n = x.shape
    quarter = m // 4
    rows = quarter // C

    def body(
        x_hbm, out_ref, xmine, xrecv,
        load_sem,
        xs, xr, ys, yr, zs, zr, fys, fyr, fzs, fzr,
    ):
        my_x = lax.axis_index("x")
        my_y = lax.axis_index("y")
        my_z = lax.axis_index("z")
        xp = (1 - my_x, my_y, my_z)
        yp = (my_x, 1 - my_y, my_z)
        zp = (my_x, my_y, 1 - my_z)

        qi = 2 * my_y + my_z
        base = qi * quarter
        base_y = (2 * (1 - my_y) + my_z) * quarter
        base_z = (2 * my_y + (1 - my_z)) * quarter

        load = pltpu.make_async_copy(
            x_hbm.at[pl.ds(base, quarter), :], xmine, load_sem
        )
        load.start()

        barrier_sem = pltpu.get_barrier_semaphore()
        pl.semaphore_signal(
            barrier_sem, inc=3,
            device_id=xp, device_id_type=pl.DeviceIdType.MESH,
        )
        for nbr in (yp, zp):
            pl.semaphore_signal(
                barrier_sem, inc=1,
                device_id=nbr, device_id_type=pl.DeviceIdType.MESH,
            )
        pl.semaphore_wait(barrier_sem, 3)
        load.wait()

        x_rdmas = []
        for c in range(C):
            sl = pl.ds(c * rows, rows)
            r = pltpu.make_async_remote_copy(
                src_ref=xmine.at[sl, :],
                dst_ref=xrecv.at[sl, :],
                send_sem=xs.at[c], recv_sem=xr.at[c],
                device_id=xp, device_id_type=pl.DeviceIdType.MESH,
            )
            r.start()
            x_rdmas.append(r)

        pl.semaphore_wait(barrier_sem, 2)

        send_rdmas = []
        for c in range(C):
            x_rdmas[c].wait_recv()
            lsl = pl.ds(c * rows, rows)
            gsl = pl.ds(base + c * rows, rows)
            out_ref[gsl, :] = xmine[lsl, :] + xrecv[lsl, :]
            for tgt, ss, rs in ((yp, ys, yr), (zp, zs, zr)):
                r = pltpu.make_async_remote_copy(
                    src_ref=out_ref.at[gsl, :],
                    dst_ref=out_ref.at[gsl, :],
                    send_sem=ss.at[c], recv_sem=rs.at[c],
                    device_id=tgt, device_id_type=pl.DeviceIdType.MESH,
                )
                r.start()
                send_rdmas.append(r)

        fwd_rdmas = []
        for c in range(H):
            zr_wait = pltpu.make_async_remote_copy(
                src_ref=out_ref.at[pl.ds(base_z + c * rows, rows), :],
                dst_ref=out_ref.at[pl.ds(base_z + c * rows, rows), :],
                send_sem=zs.at[c], recv_sem=zr.at[c],
                device_id=zp, device_id_type=pl.DeviceIdType.MESH,
            )
            zr_wait.wait_recv()
            gsl = pl.ds(base_z + c * rows, rows)
            r = pltpu.make_async_remote_copy(
                src_ref=out_ref.at[gsl, :],
                dst_ref=out_ref.at[gsl, :],
                send_sem=fys.at[c], recv_sem=fyr.at[c],
                device_id=yp, device_id_type=pl.DeviceIdType.MESH,
            )
            r.start()
            fwd_rdmas.append(r)
        for c in range(H, C):
            yr_wait = pltpu.make_async_remote_copy(
                src_ref=out_ref.at[pl.ds(base_y + c * rows, rows), :],
                dst_ref=out_ref.at[pl.ds(base_y + c * rows, rows), :],
                send_sem=ys.at[c], recv_sem=yr.at[c],
                device_id=yp, device_id_type=pl.DeviceIdType.MESH,
            )
            yr_wait.wait_recv()
            gsl = pl.ds(base_y + c * rows, rows)
            r = pltpu.make_async_remote_copy(
                src_ref=out_ref.at[gsl, :],
                dst_ref=out_ref.at[gsl, :],
                send_sem=fzs.at[c - H], recv_sem=fzr.at[c - H],
                device_id=zp, device_id_type=pl.DeviceIdType.MESH,
            )
            r.start()
            fwd_rdmas.append(r)

        for c in range(H):
            pltpu.make_async_remote_copy(
                src_ref=out_ref.at[pl.ds(base_y + c * rows, rows), :],
                dst_ref=out_ref.at[pl.ds(base_y + c * rows, rows), :],
                send_sem=ys.at[c], recv_sem=yr.at[c],
                device_id=yp, device_id_type=pl.DeviceIdType.MESH,
            ).wait_recv()
        for c in range(H, C):
            pltpu.make_async_remote_copy(
                src_ref=out_ref.at[pl.ds(base_z + c * rows, rows), :],
                dst_ref=out_ref.at[pl.ds(base_z + c * rows, rows), :],
                send_sem=zs.at[c], recv_sem=zr.at[c],
                device_id=zp, device_id_type=pl.DeviceIdType.MESH,
            ).wait_recv()
        for r in fwd_rdmas:
            r.wait_recv()
            r.wait_send()
        for r in x_rdmas:
            r.wait_send()
        for r in send_rdmas:
            r.wait_send()

    return pl.pallas_call(
        body,
        out_shape=jax.ShapeDtypeStruct((m, n), x.dtype),
        in_specs=[pl.BlockSpec(memory_space=pltpu.MemorySpace.HBM)],
        out_specs=pl.BlockSpec(memory_space=pltpu.VMEM),
        scratch_shapes=[
            pltpu.VMEM((quarter, n), x.dtype),
            pltpu.VMEM((quarter, n), x.dtype),
            pltpu.SemaphoreType.DMA,
            pltpu.SemaphoreType.DMA((C,)),
            pltpu.SemaphoreType.DMA((C,)),
            pltpu.SemaphoreType.DMA((C,)),
            pltpu.SemaphoreType.DMA((C,)),
            pltpu.SemaphoreType.DMA((C,)),
            pltpu.SemaphoreType.DMA((C,)),
            pltpu.SemaphoreType.DMA((H,)),
            pltpu.SemaphoreType.DMA((H,)),
            pltpu.SemaphoreType.DMA((H,)),
            pltpu.SemaphoreType.DMA((H,)),
        ],
        compiler_params=pltpu.CompilerParams(collective_id=0),
    )(x)
